# baseline (speedup 1.0000x reference)
import functools

import jax
import jax.numpy as jnp
from jax.experimental import pallas as pl
from jax.experimental.pallas import tpu as pltpu

DEPTH = 3


def _mlp_body(hf_ref, lng_ref, lnb_ref, w1_ref, b1_ref, w2_ref, b2_ref, out_ref):
    x = hf_ref[...]
    mu = jnp.mean(x, axis=-1, keepdims=True)
    var = jnp.mean((x - mu) ** 2, axis=-1, keepdims=True)
    x = (x - mu) * jax.lax.rsqrt(var + 1e-5) * lng_ref[...] + lnb_ref[...]
    x1 = jax.nn.relu(jnp.dot(x, w1_ref[...], preferred_element_type=jnp.float32) + b1_ref[...])
    out_ref[...] = jnp.dot(x1, w2_ref[...], preferred_element_type=jnp.float32) + b2_ref[...]


def _fusion_mlp(h_fused, ln_g, ln_b, W1, b1, W2, b2):
    n, d = h_fused.shape
    d_hid = W1.shape[1]
    blk = 200
    W2p = jnp.zeros((d_hid, 128), W2.dtype).at[:, :1].set(W2)
    b2p = jnp.zeros((128,), b2.dtype).at[:1].set(b2)
    out = pl.pallas_call(
        _mlp_body,
        grid=(n // blk,),
        in_specs=[
            pl.BlockSpec((blk, d), lambda i: (i, 0)),
            pl.BlockSpec((d,), lambda i: (0,)),
            pl.BlockSpec((d,), lambda i: (0,)),
            pl.BlockSpec((d, d_hid), lambda i: (0, 0)),
            pl.BlockSpec((d_hid,), lambda i: (0,)),
            pl.BlockSpec((d_hid, 128), lambda i: (0, 0)),
            pl.BlockSpec((128,), lambda i: (0,)),
        ],
        out_specs=pl.BlockSpec((blk, 128), lambda i: (i, 0)),
        out_shape=jax.ShapeDtypeStruct((n, 128), jnp.float32),
    )(h_fused, ln_g, ln_b, W1, b1, W2p, b2p)
    return out[:, :1]


def kernel(V, E, edge_index, rev_edge_index, batch, V_d, W_i, W_h, W_o, b_o,
           w_g, b_g, ln_g, ln_b, W1, b1, W2, b2):
    src = edge_index[0]
    dst = edge_index[1]
    H0 = jnp.concatenate([V[src], E], axis=1) @ W_i
    H = jax.nn.relu(H0)
    for _ in range(1, DEPTH):
        M_node = jax.ops.segment_sum(H, dst, num_segments=V.shape[0])
        M = M_node[src] - H[rev_edge_index]
        H = jax.nn.relu(H0 + M @ W_h)
    M_v = jax.ops.segment_sum(H, dst, num_segments=V.shape[0])
    H_v = jax.nn.relu(jnp.concatenate([V, M_v], axis=1) @ W_o + b_o)
    scores = jax.nn.sigmoid(H_v @ w_g + b_g)
    weighted = scores * H_v
    h_graph = jax.ops.segment_sum(weighted, batch, num_segments=V_d.shape[0])
    h_fused = jnp.concatenate([h_graph, V_d], axis=-1)
    return _fusion_mlp(h_fused, ln_g, ln_b, W1, b1, W2, b2)


# speedup vs baseline: 1.8318x; 1.8318x over previous
import functools

import jax
import jax.numpy as jnp
from jax import lax
from jax.experimental import pallas as pl
from jax.experimental.pallas import tpu as pltpu
from jax.experimental.pallas import tpu_sc as plsc

DEPTH = 3
N_EDGE_BLK = 4000
N_NODE_BLK = 2000


def _sc_mesh():
    return plsc.VectorSubcoreMesh(core_axis_name="c", subcore_axis_name="s")


def _sc_gather_rows(table, idx, window=128):
    B = idx.shape[0]
    D = table.shape[1]
    idx2 = idx.reshape(1, B)

    @functools.partial(
        pl.kernel,
        out_type=jax.ShapeDtypeStruct((B, D), table.dtype),
        mesh=_sc_mesh(),
    )
    def k(table_hbm, i_hbm, o_hbm):
        def body(i_vmem, o_vmem):
            pltpu.sync_copy(table_hbm.at[i_vmem.at[0]], o_vmem)

        pltpu.emit_pipeline(
            body,
            grid=(B // window,),
            in_specs=[pl.BlockSpec((1, window), lambda i: (0, i))],
            out_specs=[pl.BlockSpec((window, D), lambda i: (i, 0))],
            core_axis_name=("c", "s"),
            dimension_semantics=(pltpu.PARALLEL,),
        )(i_hbm, o_hbm)

    return k(table, idx2)


def _sc_gather_rows2(table_a, idx_a, table_b, idx_b, window=128):
    B = idx_a.shape[0]
    D = table_a.shape[1]
    ia2 = idx_a.reshape(1, B)
    ib2 = idx_b.reshape(1, B)

    @functools.partial(
        pl.kernel,
        out_type=(
            jax.ShapeDtypeStruct((B, D), table_a.dtype),
            jax.ShapeDtypeStruct((B, D), table_b.dtype),
        ),
        mesh=_sc_mesh(),
    )
    def k(ta_hbm, ia_hbm, tb_hbm, ib_hbm, oa_hbm, ob_hbm):
        def body(ia_vmem, ib_vmem, oa_vmem, ob_vmem):
            pltpu.sync_copy(ta_hbm.at[ia_vmem.at[0]], oa_vmem)
            pltpu.sync_copy(tb_hbm.at[ib_vmem.at[0]], ob_vmem)

        pltpu.emit_pipeline(
            body,
            grid=(B // window,),
            in_specs=[
                pl.BlockSpec((1, window), lambda i: (0, i)),
                pl.BlockSpec((1, window), lambda i: (0, i)),
            ],
            out_specs=[
                pl.BlockSpec((window, D), lambda i: (i, 0)),
                pl.BlockSpec((window, D), lambda i: (i, 0)),
            ],
            core_axis_name=("c", "s"),
            dimension_semantics=(pltpu.PARALLEL,),
        )(ia_hbm, ib_hbm, oa_hbm, ob_hbm)

    return k(table_a, ia2, table_b, ib2)


def _node_proj_body(v_ref, w_ref, out_ref):
    out_ref[...] = jnp.dot(v_ref[...], w_ref[...], preferred_element_type=jnp.float32)


def _node_proj(V, W):
    n, d = V.shape
    dh = W.shape[1]
    return pl.pallas_call(
        _node_proj_body,
        grid=(n // N_NODE_BLK,),
        in_specs=[
            pl.BlockSpec((N_NODE_BLK, d), lambda i: (i, 0)),
            pl.BlockSpec((d, dh), lambda i: (0, 0)),
        ],
        out_specs=pl.BlockSpec((N_NODE_BLK, dh), lambda i: (i, 0)),
        out_shape=jax.ShapeDtypeStruct((n, dh), jnp.float32),
    )(V, W)


def _h0_body(s_ref, e_ref, w_ref, h0_ref, h1_ref):
    h0 = s_ref[...] + jnp.dot(e_ref[...], w_ref[...], preferred_element_type=jnp.float32)
    h0_ref[...] = h0
    h1_ref[...] = jnp.maximum(h0, 0.0)


def _edge_init(S, E, W_ie):
    ne, dh = S.shape
    de = E.shape[1]
    return pl.pallas_call(
        _h0_body,
        grid=(ne // N_EDGE_BLK,),
        in_specs=[
            pl.BlockSpec((N_EDGE_BLK, dh), lambda i: (i, 0)),
            pl.BlockSpec((N_EDGE_BLK, de), lambda i: (i, 0)),
            pl.BlockSpec((de, dh), lambda i: (0, 0)),
        ],
        out_specs=[
            pl.BlockSpec((N_EDGE_BLK, dh), lambda i: (i, 0)),
            pl.BlockSpec((N_EDGE_BLK, dh), lambda i: (i, 0)),
        ],
        out_shape=[
            jax.ShapeDtypeStruct((ne, dh), jnp.float32),
            jax.ShapeDtypeStruct((ne, dh), jnp.float32),
        ],
    )(S, E, W_ie)


def _combine_body(h0_ref, mg_ref, hg_ref, w_ref, out_ref):
    m = mg_ref[...] - hg_ref[...]
    out_ref[...] = jnp.maximum(
        h0_ref[...] + jnp.dot(m, w_ref[...], preferred_element_type=jnp.float32), 0.0
    )


def _edge_combine(H0, Mg, Hg, W_h):
    ne, dh = H0.shape
    return pl.pallas_call(
        _combine_body,
        grid=(ne // N_EDGE_BLK,),
        in_specs=[
            pl.BlockSpec((N_EDGE_BLK, dh), lambda i: (i, 0)),
            pl.BlockSpec((N_EDGE_BLK, dh), lambda i: (i, 0)),
            pl.BlockSpec((N_EDGE_BLK, dh), lambda i: (i, 0)),
            pl.BlockSpec((dh, dh), lambda i: (0, 0)),
        ],
        out_specs=pl.BlockSpec((N_EDGE_BLK, dh), lambda i: (i, 0)),
        out_shape=jax.ShapeDtypeStruct((ne, dh), jnp.float32),
    )(H0, Mg, Hg, W_h)


def _hv_body(v_ref, mv_ref, wo1_ref, wo2_ref, bo_ref, wg_ref, bg_ref, out_ref):
    hv = jnp.dot(v_ref[...], wo1_ref[...], preferred_element_type=jnp.float32)
    hv += jnp.dot(mv_ref[...], wo2_ref[...], preferred_element_type=jnp.float32)
    hv = jnp.maximum(hv + bo_ref[...], 0.0)
    s = jnp.sum(hv * wg_ref[...], axis=1, keepdims=True) + bg_ref[...]
    out_ref[...] = jax.nn.sigmoid(s) * hv


def _node_out(V, M_v, W_o1, W_o2, b_o, wg_row, b_g):
    n, dv = V.shape
    dh = M_v.shape[1]
    return pl.pallas_call(
        _hv_body,
        grid=(n // N_NODE_BLK,),
        in_specs=[
            pl.BlockSpec((N_NODE_BLK, dv), lambda i: (i, 0)),
            pl.BlockSpec((N_NODE_BLK, dh), lambda i: (i, 0)),
            pl.BlockSpec((dv, dh), lambda i: (0, 0)),
            pl.BlockSpec((dh, dh), lambda i: (0, 0)),
            pl.BlockSpec((dh,), lambda i: (0,)),
            pl.BlockSpec((1, dh), lambda i: (0, 0)),
            pl.BlockSpec((1, 1), lambda i: (0, 0)),
        ],
        out_specs=pl.BlockSpec((N_NODE_BLK, dh), lambda i: (i, 0)),
        out_shape=jax.ShapeDtypeStruct((n, dh), jnp.float32),
    )(V, M_v, W_o1, W_o2, b_o, wg_row, b_g)


def _mlp_body(hg_ref, vd_ref, lng_ref, lnb_ref, w1_ref, b1_ref, w2_ref, b2_ref, out_ref):
    x = jnp.concatenate([hg_ref[...], vd_ref[...]], axis=1)
    mu = jnp.mean(x, axis=-1, keepdims=True)
    var = jnp.mean((x - mu) ** 2, axis=-1, keepdims=True)
    x = (x - mu) * lax.rsqrt(var + 1e-5) * lng_ref[...] + lnb_ref[...]
    x1 = jnp.maximum(jnp.dot(x, w1_ref[...], preferred_element_type=jnp.float32) + b1_ref[...], 0.0)
    out_ref[...] = jnp.dot(x1, w2_ref[...], preferred_element_type=jnp.float32) + b2_ref[...]


def _fusion_mlp(h_graph, V_d, ln_g, ln_b, W1, b1, W2, b2):
    n, dh = h_graph.shape
    dlm = V_d.shape[1]
    d = dh + dlm
    d_hid = W1.shape[1]
    blk = 200
    W2p = jnp.zeros((d_hid, 128), W2.dtype).at[:, :1].set(W2)
    b2p = jnp.zeros((128,), b2.dtype).at[:1].set(b2)
    out = pl.pallas_call(
        _mlp_body,
        grid=(n // blk,),
        in_specs=[
            pl.BlockSpec((blk, dh), lambda i: (i, 0)),
            pl.BlockSpec((blk, dlm), lambda i: (i, 0)),
            pl.BlockSpec((d,), lambda i: (0,)),
            pl.BlockSpec((d,), lambda i: (0,)),
            pl.BlockSpec((d, d_hid), lambda i: (0, 0)),
            pl.BlockSpec((d_hid,), lambda i: (0,)),
            pl.BlockSpec((d_hid, 128), lambda i: (0, 0)),
            pl.BlockSpec((128,), lambda i: (0,)),
        ],
        out_specs=pl.BlockSpec((blk, 128), lambda i: (i, 0)),
        out_shape=jax.ShapeDtypeStruct((n, 128), jnp.float32),
    )(h_graph, V_d, ln_g, ln_b, W1, b1, W2p, b2p)
    return out[:, :1]


def kernel(V, E, edge_index, rev_edge_index, batch, V_d, W_i, W_h, W_o, b_o,
           w_g, b_g, ln_g, ln_b, W1, b1, W2, b2):
    n_nodes, d_v = V.shape
    src = edge_index[0]
    dst = edge_index[1]

    W_iv = W_i[:d_v]
    W_ie = W_i[d_v:]
    W_o1 = W_o[:d_v]
    W_o2 = W_o[d_v:]

    P = _node_proj(V, W_iv)
    S = _sc_gather_rows(P, src)
    H0, H = _edge_init(S, E, W_ie)

    for _ in range(1, DEPTH):
        M_node = jax.ops.segment_sum(H, dst, num_segments=n_nodes)
        Mg, Hg = _sc_gather_rows2(M_node, src, H, rev_edge_index)
        H = _edge_combine(H0, Mg, Hg, W_h)

    M_v = jax.ops.segment_sum(H, dst, num_segments=n_nodes)
    weighted = _node_out(V, M_v, W_o1, W_o2, b_o, w_g.reshape(1, -1), b_g.reshape(1, 1))
    h_graph = jax.ops.segment_sum(weighted, batch, num_segments=V_d.shape[0])
    return _fusion_mlp(h_graph, V_d, ln_g, ln_b, W1, b1, W2, b2)


# speedup vs baseline: 8.0602x; 4.4003x over previous
import functools

import jax
import jax.numpy as jnp
from jax import lax
from jax.experimental import pallas as pl
from jax.experimental.pallas import tpu as pltpu
from jax.experimental.pallas import tpu_sc as plsc

DEPTH = 3
N_EDGE_BLK = 4000
N_NODE_BLK = 2000


def _sc_mesh():
    return plsc.VectorSubcoreMesh(core_axis_name="c", subcore_axis_name="s")


def _sc_gather_rows(table, idx, window=128):
    B = idx.shape[0]
    D = table.shape[1]
    idx2 = idx.reshape(1, B)

    @functools.partial(
        pl.kernel,
        out_type=jax.ShapeDtypeStruct((B, D), table.dtype),
        mesh=_sc_mesh(),
    )
    def k(table_hbm, i_hbm, o_hbm):
        def body(i_vmem, o_vmem):
            pltpu.sync_copy(table_hbm.at[i_vmem.at[0]], o_vmem)

        pltpu.emit_pipeline(
            body,
            grid=(B // window,),
            in_specs=[pl.BlockSpec((1, window), lambda i: (0, i))],
            out_specs=[pl.BlockSpec((window, D), lambda i: (i, 0))],
            core_axis_name=("c", "s"),
            dimension_semantics=(pltpu.PARALLEL,),
        )(i_hbm, o_hbm)

    return k(table, idx2)


def _sc_gather_rows2(table_a, idx_a, table_b, idx_b, window=128):
    B = idx_a.shape[0]
    D = table_a.shape[1]
    ia2 = idx_a.reshape(1, B)
    ib2 = idx_b.reshape(1, B)

    @functools.partial(
        pl.kernel,
        out_type=(
            jax.ShapeDtypeStruct((B, D), table_a.dtype),
            jax.ShapeDtypeStruct((B, D), table_b.dtype),
        ),
        mesh=_sc_mesh(),
    )
    def k(ta_hbm, ia_hbm, tb_hbm, ib_hbm, oa_hbm, ob_hbm):
        def body(ia_vmem, ib_vmem, oa_vmem, ob_vmem):
            pltpu.sync_copy(ta_hbm.at[ia_vmem.at[0]], oa_vmem)
            pltpu.sync_copy(tb_hbm.at[ib_vmem.at[0]], ob_vmem)

        pltpu.emit_pipeline(
            body,
            grid=(B // window,),
            in_specs=[
                pl.BlockSpec((1, window), lambda i: (0, i)),
                pl.BlockSpec((1, window), lambda i: (0, i)),
            ],
            out_specs=[
                pl.BlockSpec((window, D), lambda i: (i, 0)),
                pl.BlockSpec((window, D), lambda i: (i, 0)),
            ],
            core_axis_name=("c", "s"),
            dimension_semantics=(pltpu.PARALLEL,),
        )(ia_hbm, ib_hbm, oa_hbm, ob_hbm)

    return k(table_a, ia2, table_b, ib2)


def _sc_segsum_partials(X, ids, n_out, n_quarters, window=128):
    ne, D = X.shape
    DQ = D // n_quarters
    rpt = ((n_out + 15) // 16 + 7) // 8 * 8
    n_out = rpt * 16
    Z = jnp.zeros((n_out, D), jnp.float32)

    @functools.partial(
        pl.kernel,
        out_type=jax.ShapeDtypeStruct((2, n_out, D), jnp.float32),
        mesh=_sc_mesh(),
        scratch_types=[pltpu.VMEM_SHARED((n_out, DQ), jnp.float32)],
        compiler_params=pltpu.CompilerParams(use_tc_tiling_on_sc=False),
    )
    def k(x_hbm, i_hbm, z_hbm, o_hbm, acc_sh):
        c = lax.axis_index("c")
        s = lax.axis_index("s")
        row0 = s * rpt
        for q in range(n_quarters):
            pltpu.sync_copy(z_hbm.at[pl.ds(row0, rpt), pl.ds(0, DQ)],
                            acc_sh.at[pl.ds(row0, rpt), :])
            plsc.subcore_barrier()

            def body(i_vmem, x_vmem):
                pltpu.sync_copy(x_vmem, acc_sh.at[i_vmem], add=True)

            pltpu.emit_pipeline(
                body,
                grid=(ne // window,),
                in_specs=[
                    pl.BlockSpec((window,), lambda i: (i,)),
                    pl.BlockSpec((window, DQ), lambda i, q=q: (i, q)),
                ],
                core_axis_name=("c", "s"),
                dimension_semantics=(pltpu.PARALLEL,),
            )(i_hbm, x_hbm)
            plsc.subcore_barrier()
            pltpu.sync_copy(acc_sh.at[pl.ds(row0, rpt), :],
                            o_hbm.at[c, pl.ds(row0, rpt), pl.ds(q * DQ, DQ)])
            plsc.subcore_barrier()

    return k(X, ids, Z)


def _merge_body(p0_ref, p1_ref, out_ref):
    out_ref[...] = p0_ref[...] + p1_ref[...]


def _merge_add(p0, p1):
    n, d = p0.shape
    return pl.pallas_call(
        _merge_body,
        grid=(pl.cdiv(n, N_NODE_BLK),),
        in_specs=[
            pl.BlockSpec((N_NODE_BLK, d), lambda i: (i, 0)),
            pl.BlockSpec((N_NODE_BLK, d), lambda i: (i, 0)),
        ],
        out_specs=pl.BlockSpec((N_NODE_BLK, d), lambda i: (i, 0)),
        out_shape=jax.ShapeDtypeStruct((n, d), jnp.float32),
    )(p0, p1)


def _node_proj_body(v_ref, w_ref, out_ref):
    out_ref[...] = jnp.dot(v_ref[...], w_ref[...], preferred_element_type=jnp.float32)


def _node_proj(V, W):
    n, d = V.shape
    dh = W.shape[1]
    return pl.pallas_call(
        _node_proj_body,
        grid=(n // N_NODE_BLK,),
        in_specs=[
            pl.BlockSpec((N_NODE_BLK, d), lambda i: (i, 0)),
            pl.BlockSpec((d, dh), lambda i: (0, 0)),
        ],
        out_specs=pl.BlockSpec((N_NODE_BLK, dh), lambda i: (i, 0)),
        out_shape=jax.ShapeDtypeStruct((n, dh), jnp.float32),
    )(V, W)


def _h0_body(s_ref, e_ref, w_ref, h0_ref, h1_ref):
    h0 = s_ref[...] + jnp.dot(e_ref[...], w_ref[...], preferred_element_type=jnp.float32)
    h0_ref[...] = h0
    h1_ref[...] = jnp.maximum(h0, 0.0)


def _edge_init(S, E, W_ie):
    ne, dh = S.shape
    de = E.shape[1]
    return pl.pallas_call(
        _h0_body,
        grid=(ne // N_EDGE_BLK,),
        in_specs=[
            pl.BlockSpec((N_EDGE_BLK, dh), lambda i: (i, 0)),
            pl.BlockSpec((N_EDGE_BLK, de), lambda i: (i, 0)),
            pl.BlockSpec((de, dh), lambda i: (0, 0)),
        ],
        out_specs=[
            pl.BlockSpec((N_EDGE_BLK, dh), lambda i: (i, 0)),
            pl.BlockSpec((N_EDGE_BLK, dh), lambda i: (i, 0)),
        ],
        out_shape=[
            jax.ShapeDtypeStruct((ne, dh), jnp.float32),
            jax.ShapeDtypeStruct((ne, dh), jnp.float32),
        ],
    )(S, E, W_ie)


def _combine_body(h0_ref, mg_ref, hg_ref, w_ref, out_ref):
    m = mg_ref[...] - hg_ref[...]
    out_ref[...] = jnp.maximum(
        h0_ref[...] + jnp.dot(m, w_ref[...], preferred_element_type=jnp.float32), 0.0
    )


def _edge_combine(H0, Mg, Hg, W_h):
    ne, dh = H0.shape
    return pl.pallas_call(
        _combine_body,
        grid=(ne // N_EDGE_BLK,),
        in_specs=[
            pl.BlockSpec((N_EDGE_BLK, dh), lambda i: (i, 0)),
            pl.BlockSpec((N_EDGE_BLK, dh), lambda i: (i, 0)),
            pl.BlockSpec((N_EDGE_BLK, dh), lambda i: (i, 0)),
            pl.BlockSpec((dh, dh), lambda i: (0, 0)),
        ],
        out_specs=pl.BlockSpec((N_EDGE_BLK, dh), lambda i: (i, 0)),
        out_shape=jax.ShapeDtypeStruct((ne, dh), jnp.float32),
    )(H0, Mg, Hg, W_h)


def _hv_body(v_ref, mv0_ref, mv1_ref, wo1_ref, wo2_ref, bo_ref, wg_ref, bg_ref, out_ref):
    hv = jnp.dot(v_ref[...], wo1_ref[...], preferred_element_type=jnp.float32)
    hv += jnp.dot(mv0_ref[...] + mv1_ref[...], wo2_ref[...], preferred_element_type=jnp.float32)
    hv = jnp.maximum(hv + bo_ref[...], 0.0)
    s = jnp.sum(hv * wg_ref[...], axis=1, keepdims=True) + bg_ref[...]
    out_ref[...] = jax.nn.sigmoid(s) * hv


def _node_out(V, Mv0, Mv1, W_o1, W_o2, b_o, wg_row, b_g):
    n, dv = V.shape
    dh = Mv0.shape[1]
    return pl.pallas_call(
        _hv_body,
        grid=(n // N_NODE_BLK,),
        in_specs=[
            pl.BlockSpec((N_NODE_BLK, dv), lambda i: (i, 0)),
            pl.BlockSpec((N_NODE_BLK, dh), lambda i: (i, 0)),
            pl.BlockSpec((N_NODE_BLK, dh), lambda i: (i, 0)),
            pl.BlockSpec((dv, dh), lambda i: (0, 0)),
            pl.BlockSpec((dh, dh), lambda i: (0, 0)),
            pl.BlockSpec((dh,), lambda i: (0,)),
            pl.BlockSpec((1, dh), lambda i: (0, 0)),
            pl.BlockSpec((1, 1), lambda i: (0, 0)),
        ],
        out_specs=pl.BlockSpec((N_NODE_BLK, dh), lambda i: (i, 0)),
        out_shape=jax.ShapeDtypeStruct((n, dh), jnp.float32),
    )(V, Mv0, Mv1, W_o1, W_o2, b_o, wg_row, b_g)


def _mlp_body(hg0_ref, hg1_ref, vd_ref, lng_ref, lnb_ref, w1_ref, b1_ref, w2_ref, b2_ref, out_ref):
    x = jnp.concatenate([hg0_ref[...] + hg1_ref[...], vd_ref[...]], axis=1)
    mu = jnp.mean(x, axis=-1, keepdims=True)
    var = jnp.mean((x - mu) ** 2, axis=-1, keepdims=True)
    x = (x - mu) * lax.rsqrt(var + 1e-5) * lng_ref[...] + lnb_ref[...]
    x1 = jnp.maximum(jnp.dot(x, w1_ref[...], preferred_element_type=jnp.float32) + b1_ref[...], 0.0)
    out_ref[...] = jnp.dot(x1, w2_ref[...], preferred_element_type=jnp.float32) + b2_ref[...]


def _fusion_mlp(hg0, hg1, V_d, ln_g, ln_b, W1, b1, W2, b2):
    n = V_d.shape[0]
    dh = hg0.shape[1]
    dlm = V_d.shape[1]
    d = dh + dlm
    d_hid = W1.shape[1]
    blk = 200
    W2p = jnp.zeros((d_hid, 128), W2.dtype).at[:, :1].set(W2)
    b2p = jnp.zeros((128,), b2.dtype).at[:1].set(b2)
    out = pl.pallas_call(
        _mlp_body,
        grid=(n // blk,),
        in_specs=[
            pl.BlockSpec((blk, dh), lambda i: (i, 0)),
            pl.BlockSpec((blk, dh), lambda i: (i, 0)),
            pl.BlockSpec((blk, dlm), lambda i: (i, 0)),
            pl.BlockSpec((d,), lambda i: (0,)),
            pl.BlockSpec((d,), lambda i: (0,)),
            pl.BlockSpec((d, d_hid), lambda i: (0, 0)),
            pl.BlockSpec((d_hid,), lambda i: (0,)),
            pl.BlockSpec((d_hid, 128), lambda i: (0, 0)),
            pl.BlockSpec((128,), lambda i: (0,)),
        ],
        out_specs=pl.BlockSpec((blk, 128), lambda i: (i, 0)),
        out_shape=jax.ShapeDtypeStruct((n, 128), jnp.float32),
    )(hg0, hg1, V_d, ln_g, ln_b, W1, b1, W2p, b2p)
    return out[:, :1]


def kernel(V, E, edge_index, rev_edge_index, batch, V_d, W_i, W_h, W_o, b_o,
           w_g, b_g, ln_g, ln_b, W1, b1, W2, b2):
    n_nodes, d_v = V.shape
    src = edge_index[0]
    dst = edge_index[1]

    W_iv = W_i[:d_v]
    W_ie = W_i[d_v:]
    W_o1 = W_o[:d_v]
    W_o2 = W_o[d_v:]

    P = _node_proj(V, W_iv)
    S = _sc_gather_rows(P, src)
    H0, H = _edge_init(S, E, W_ie)

    for _ in range(1, DEPTH):
        Mp = _sc_segsum_partials(H, dst, n_nodes, 4)
        M_node = _merge_add(Mp[0], Mp[1])
        Mg, Hg = _sc_gather_rows2(M_node, src, H, rev_edge_index)
        H = _edge_combine(H0, Mg, Hg, W_h)

    Mvp = _sc_segsum_partials(H, dst, n_nodes, 4)
    weighted = _node_out(V, Mvp[0], Mvp[1], W_o1, W_o2, b_o,
                         w_g.reshape(1, -1), b_g.reshape(1, 1))
    Pp = _sc_segsum_partials(weighted, batch, V_d.shape[0], 1, window=80)
    return _fusion_mlp(Pp[0], Pp[1], V_d, ln_g, ln_b, W1, b1, W2, b2)


# speedup vs baseline: 11.7420x; 1.4568x over previous
import functools

import jax
import jax.numpy as jnp
from jax import lax
from jax.experimental import pallas as pl
from jax.experimental.pallas import tpu as pltpu
from jax.experimental.pallas import tpu_sc as plsc

DEPTH = 3
N_EDGE_BLK = 4000
N_NODE_BLK = 2000


def _sc_mesh():
    return plsc.VectorSubcoreMesh(core_axis_name="c", subcore_axis_name="s")


def _sc_gather_rows(table, idx, window=128):
    B = idx.shape[0]
    D = table.shape[1]
    idx2 = idx.reshape(1, B)

    @functools.partial(
        pl.kernel,
        out_type=jax.ShapeDtypeStruct((B, D), table.dtype),
        mesh=_sc_mesh(),
    )
    def k(table_hbm, i_hbm, o_hbm):
        def body(i_vmem, o_vmem):
            pltpu.sync_copy(table_hbm.at[i_vmem.at[0]], o_vmem)

        pltpu.emit_pipeline(
            body,
            grid=(B // window,),
            in_specs=[pl.BlockSpec((1, window), lambda i: (0, i))],
            out_specs=[pl.BlockSpec((window, D), lambda i: (i, 0))],
            core_axis_name=("c", "s"),
            dimension_semantics=(pltpu.PARALLEL,),
        )(i_hbm, o_hbm)

    return k(table, idx2)


def _sc_gather_small(table, idx, n_stripes=4, window=128):
    nt, D = table.shape
    B = idx.shape[0]
    DQ = D // n_stripes
    rpt = nt // 16

    @functools.partial(
        pl.kernel,
        out_type=jax.ShapeDtypeStruct((B, D), jnp.float32),
        mesh=_sc_mesh(),
        scratch_types=[pltpu.VMEM_SHARED((nt, DQ), jnp.float32)],
        compiler_params=pltpu.CompilerParams(use_tc_tiling_on_sc=False),
    )
    def k(t_hbm, i_hbm, o_hbm, tab_sh):
        s = lax.axis_index("s")
        row0 = s * rpt
        for q in range(n_stripes):
            pltpu.sync_copy(t_hbm.at[pl.ds(row0, rpt), pl.ds(q * DQ, DQ)],
                            tab_sh.at[pl.ds(row0, rpt), :])
            plsc.subcore_barrier()

            def body(i_vmem, o_vmem):
                pltpu.sync_copy(tab_sh.at[i_vmem], o_vmem)

            pltpu.emit_pipeline(
                body,
                grid=(B // window,),
                in_specs=[pl.BlockSpec((window,), lambda i: (i,))],
                out_specs=[pl.BlockSpec((window, DQ), lambda i, q=q: (i, q))],
                core_axis_name=("c", "s"),
                dimension_semantics=(pltpu.PARALLEL,),
            )(i_hbm, o_hbm)
            plsc.subcore_barrier()

    return k(table, idx)


def _sc_segsum_partials(X, ids, n_out, n_quarters, window=128):
    ne, D = X.shape
    DQ = D // n_quarters
    rpt = ((n_out + 15) // 16 + 7) // 8 * 8
    n_out = rpt * 16
    Z = jnp.zeros((n_out, D), jnp.float32)

    @functools.partial(
        pl.kernel,
        out_type=jax.ShapeDtypeStruct((2, n_out, D), jnp.float32),
        mesh=_sc_mesh(),
        scratch_types=[pltpu.VMEM_SHARED((n_out, DQ), jnp.float32)],
        compiler_params=pltpu.CompilerParams(use_tc_tiling_on_sc=False),
    )
    def k(x_hbm, i_hbm, z_hbm, o_hbm, acc_sh):
        c = lax.axis_index("c")
        s = lax.axis_index("s")
        row0 = s * rpt
        for q in range(n_quarters):
            pltpu.sync_copy(z_hbm.at[pl.ds(row0, rpt), pl.ds(0, DQ)],
                            acc_sh.at[pl.ds(row0, rpt), :])
            plsc.subcore_barrier()

            def body(i_vmem, x_vmem):
                pltpu.sync_copy(x_vmem, acc_sh.at[i_vmem], add=True)

            pltpu.emit_pipeline(
                body,
                grid=(ne // window,),
                in_specs=[
                    pl.BlockSpec((window,), lambda i: (i,)),
                    pl.BlockSpec((window, DQ), lambda i, q=q: (i, q)),
                ],
                core_axis_name=("c", "s"),
                dimension_semantics=(pltpu.PARALLEL,),
            )(i_hbm, x_hbm)
            plsc.subcore_barrier()
            pltpu.sync_copy(acc_sh.at[pl.ds(row0, rpt), :],
                            o_hbm.at[c, pl.ds(row0, rpt), pl.ds(q * DQ, DQ)])
            plsc.subcore_barrier()

    return k(X, ids, Z)


def _merge_body(p0_ref, p1_ref, out_ref):
    out_ref[...] = p0_ref[...] + p1_ref[...]


def _merge_add(p0, p1):
    n, d = p0.shape
    return pl.pallas_call(
        _merge_body,
        grid=(pl.cdiv(n, N_NODE_BLK),),
        in_specs=[
            pl.BlockSpec((N_NODE_BLK, d), lambda i: (i, 0)),
            pl.BlockSpec((N_NODE_BLK, d), lambda i: (i, 0)),
        ],
        out_specs=pl.BlockSpec((N_NODE_BLK, d), lambda i: (i, 0)),
        out_shape=jax.ShapeDtypeStruct((n, d), jnp.float32),
    )(p0, p1)


def _node_proj_body(v_ref, w_ref, out_ref):
    out_ref[...] = jnp.dot(v_ref[...], w_ref[...], preferred_element_type=jnp.float32)


def _node_proj(V, W, n_pad):
    n, d = V.shape
    dh = W.shape[1]
    return pl.pallas_call(
        _node_proj_body,
        grid=(pl.cdiv(n_pad, N_NODE_BLK),),
        in_specs=[
            pl.BlockSpec((N_NODE_BLK, d), lambda i: (i, 0)),
            pl.BlockSpec((d, dh), lambda i: (0, 0)),
        ],
        out_specs=pl.BlockSpec((N_NODE_BLK, dh), lambda i: (i, 0)),
        out_shape=jax.ShapeDtypeStruct((n_pad, dh), jnp.float32),
    )(V, W)


def _h0_body(s_ref, e_ref, w_ref, h0_ref, h1_ref):
    h0 = s_ref[...] + jnp.dot(e_ref[...], w_ref[...], preferred_element_type=jnp.float32)
    h0_ref[...] = h0
    h1_ref[...] = jnp.maximum(h0, 0.0)


def _edge_init(S, E, W_ie):
    ne, dh = S.shape
    de = E.shape[1]
    return pl.pallas_call(
        _h0_body,
        grid=(ne // N_EDGE_BLK,),
        in_specs=[
            pl.BlockSpec((N_EDGE_BLK, dh), lambda i: (i, 0)),
            pl.BlockSpec((N_EDGE_BLK, de), lambda i: (i, 0)),
            pl.BlockSpec((de, dh), lambda i: (0, 0)),
        ],
        out_specs=[
            pl.BlockSpec((N_EDGE_BLK, dh), lambda i: (i, 0)),
            pl.BlockSpec((N_EDGE_BLK, dh), lambda i: (i, 0)),
        ],
        out_shape=[
            jax.ShapeDtypeStruct((ne, dh), jnp.float32),
            jax.ShapeDtypeStruct((ne, dh), jnp.float32),
        ],
    )(S, E, W_ie)


def _combine_body(h0_ref, mg_ref, hg_ref, w_ref, out_ref):
    m = mg_ref[...] - hg_ref[...]
    out_ref[...] = jnp.maximum(
        h0_ref[...] + jnp.dot(m, w_ref[...], preferred_element_type=jnp.float32), 0.0
    )


def _edge_combine(H0, Mg, Hg, W_h):
    ne, dh = H0.shape
    return pl.pallas_call(
        _combine_body,
        grid=(ne // N_EDGE_BLK,),
        in_specs=[
            pl.BlockSpec((N_EDGE_BLK, dh), lambda i: (i, 0)),
            pl.BlockSpec((N_EDGE_BLK, dh), lambda i: (i, 0)),
            pl.BlockSpec((N_EDGE_BLK, dh), lambda i: (i, 0)),
            pl.BlockSpec((dh, dh), lambda i: (0, 0)),
        ],
        out_specs=pl.BlockSpec((N_EDGE_BLK, dh), lambda i: (i, 0)),
        out_shape=jax.ShapeDtypeStruct((ne, dh), jnp.float32),
    )(H0, Mg, Hg, W_h)


def _hv_body(v_ref, mv0_ref, mv1_ref, wo1_ref, wo2_ref, bo_ref, wg_ref, bg_ref, out_ref):
    hv = jnp.dot(v_ref[...], wo1_ref[...], preferred_element_type=jnp.float32)
    hv += jnp.dot(mv0_ref[...] + mv1_ref[...], wo2_ref[...], preferred_element_type=jnp.float32)
    hv = jnp.maximum(hv + bo_ref[...], 0.0)
    s = jnp.sum(hv * wg_ref[...], axis=1, keepdims=True) + bg_ref[...]
    out_ref[...] = jax.nn.sigmoid(s) * hv


def _node_out(V, Mv0, Mv1, W_o1, W_o2, b_o, wg_row, b_g):
    n, dv = V.shape
    dh = Mv0.shape[1]
    return pl.pallas_call(
        _hv_body,
        grid=(n // N_NODE_BLK,),
        in_specs=[
            pl.BlockSpec((N_NODE_BLK, dv), lambda i: (i, 0)),
            pl.BlockSpec((N_NODE_BLK, dh), lambda i: (i, 0)),
            pl.BlockSpec((N_NODE_BLK, dh), lambda i: (i, 0)),
            pl.BlockSpec((dv, dh), lambda i: (0, 0)),
            pl.BlockSpec((dh, dh), lambda i: (0, 0)),
            pl.BlockSpec((dh,), lambda i: (0,)),
            pl.BlockSpec((1, dh), lambda i: (0, 0)),
            pl.BlockSpec((1, 1), lambda i: (0, 0)),
        ],
        out_specs=pl.BlockSpec((N_NODE_BLK, dh), lambda i: (i, 0)),
        out_shape=jax.ShapeDtypeStruct((n, dh), jnp.float32),
    )(V, Mv0, Mv1, W_o1, W_o2, b_o, wg_row, b_g)


def _mlp_body(hg0_ref, hg1_ref, vd_ref, lng_ref, lnb_ref, w1_ref, b1_ref, w2_ref, b2_ref, out_ref):
    x = jnp.concatenate([hg0_ref[...] + hg1_ref[...], vd_ref[...]], axis=1)
    mu = jnp.mean(x, axis=-1, keepdims=True)
    var = jnp.mean((x - mu) ** 2, axis=-1, keepdims=True)
    x = (x - mu) * lax.rsqrt(var + 1e-5) * lng_ref[...] + lnb_ref[...]
    x1 = jnp.maximum(jnp.dot(x, w1_ref[...], preferred_element_type=jnp.float32) + b1_ref[...], 0.0)
    out_ref[...] = jnp.dot(x1, w2_ref[...], preferred_element_type=jnp.float32) + b2_ref[...]


def _fusion_mlp(hg0, hg1, V_d, ln_g, ln_b, W1, b1, W2, b2):
    n = V_d.shape[0]
    dh = hg0.shape[1]
    dlm = V_d.shape[1]
    d = dh + dlm
    d_hid = W1.shape[1]
    blk = 200
    W2p = jnp.zeros((d_hid, 128), W2.dtype).at[:, :1].set(W2)
    b2p = jnp.zeros((128,), b2.dtype).at[:1].set(b2)
    out = pl.pallas_call(
        _mlp_body,
        grid=(n // blk,),
        in_specs=[
            pl.BlockSpec((blk, dh), lambda i: (i, 0)),
            pl.BlockSpec((blk, dh), lambda i: (i, 0)),
            pl.BlockSpec((blk, dlm), lambda i: (i, 0)),
            pl.BlockSpec((d,), lambda i: (0,)),
            pl.BlockSpec((d,), lambda i: (0,)),
            pl.BlockSpec((d, d_hid), lambda i: (0, 0)),
            pl.BlockSpec((d_hid,), lambda i: (0,)),
            pl.BlockSpec((d_hid, 128), lambda i: (0, 0)),
            pl.BlockSpec((128,), lambda i: (0,)),
        ],
        out_specs=pl.BlockSpec((blk, 128), lambda i: (i, 0)),
        out_shape=jax.ShapeDtypeStruct((n, 128), jnp.float32),
    )(hg0, hg1, V_d, ln_g, ln_b, W1, b1, W2p, b2p)
    return out[:, :1]


def kernel(V, E, edge_index, rev_edge_index, batch, V_d, W_i, W_h, W_o, b_o,
           w_g, b_g, ln_g, ln_b, W1, b1, W2, b2):
    n_nodes, d_v = V.shape
    src = edge_index[0]
    dst = edge_index[1]
    n_pad = ((n_nodes + 127) // 128) * 128

    W_iv = W_i[:d_v]
    W_ie = W_i[d_v:]
    W_o1 = W_o[:d_v]
    W_o2 = W_o[d_v:]

    P = _node_proj(V, W_iv, n_pad)
    S = _sc_gather_small(P, src)
    H0, H = _edge_init(S, E, W_ie)

    for _ in range(1, DEPTH):
        Mp = _sc_segsum_partials(H, dst, n_nodes, 4)
        M_node = _merge_add(Mp[0], Mp[1])
        Mg = _sc_gather_small(M_node, src)
        Hg = _sc_gather_rows(H, rev_edge_index)
        H = _edge_combine(H0, Mg, Hg, W_h)

    Mvp = _sc_segsum_partials(H, dst, n_nodes, 4)
    weighted = _node_out(V, Mvp[0], Mvp[1], W_o1, W_o2, b_o,
                         w_g.reshape(1, -1), b_g.reshape(1, 1))
    Pp = _sc_segsum_partials(weighted, batch, V_d.shape[0], 1, window=80)
    return _fusion_mlp(Pp[0], Pp[1], V_d, ln_g, ln_b, W1, b1, W2, b2)


# speedup vs baseline: 12.0856x; 1.0293x over previous
import functools

import jax
import jax.numpy as jnp
from jax import lax
from jax.experimental import pallas as pl
from jax.experimental.pallas import tpu as pltpu
from jax.experimental.pallas import tpu_sc as plsc

DEPTH = 3
N_EDGE_BLK = 4000
N_NODE_BLK = 2000


def _sc_mesh():
    return plsc.VectorSubcoreMesh(core_axis_name="c", subcore_axis_name="s")


def _sc_gather_rows(table, idx, window=128):
    B = idx.shape[0]
    D = table.shape[1]
    idx2 = idx.reshape(1, B)

    @functools.partial(
        pl.kernel,
        out_type=jax.ShapeDtypeStruct((B, D), table.dtype),
        mesh=_sc_mesh(),
    )
    def k(table_hbm, i_hbm, o_hbm):
        def body(i_vmem, o_vmem):
            pltpu.sync_copy(table_hbm.at[i_vmem.at[0]], o_vmem)

        pltpu.emit_pipeline(
            body,
            grid=(B // window,),
            in_specs=[pl.BlockSpec((1, window), lambda i: (0, i))],
            out_specs=[pl.BlockSpec((window, D), lambda i: (i, 0))],
            core_axis_name=("c", "s"),
            dimension_semantics=(pltpu.PARALLEL,),
        )(i_hbm, o_hbm)

    return k(table, idx2)


def _sc_gather_rows2(table_a, idx_a, table_b, idx_b, window=128):
    B = idx_a.shape[0]
    D = table_a.shape[1]
    ia2 = idx_a.reshape(1, B)
    ib2 = idx_b.reshape(1, B)

    @functools.partial(
        pl.kernel,
        out_type=(
            jax.ShapeDtypeStruct((B, D), table_a.dtype),
            jax.ShapeDtypeStruct((B, D), table_b.dtype),
        ),
        mesh=_sc_mesh(),
    )
    def k(ta_hbm, ia_hbm, tb_hbm, ib_hbm, oa_hbm, ob_hbm):
        def body(ia_vmem, ib_vmem, oa_vmem, ob_vmem):
            pltpu.sync_copy(ta_hbm.at[ia_vmem.at[0]], oa_vmem)
            pltpu.sync_copy(tb_hbm.at[ib_vmem.at[0]], ob_vmem)

        pltpu.emit_pipeline(
            body,
            grid=(B // window,),
            in_specs=[
                pl.BlockSpec((1, window), lambda i: (0, i)),
                pl.BlockSpec((1, window), lambda i: (0, i)),
            ],
            out_specs=[
                pl.BlockSpec((window, D), lambda i: (i, 0)),
                pl.BlockSpec((window, D), lambda i: (i, 0)),
            ],
            core_axis_name=("c", "s"),
            dimension_semantics=(pltpu.PARALLEL,),
        )(ia_hbm, ib_hbm, oa_hbm, ob_hbm)

    return k(table_a, ia2, table_b, ib2)


def _sc_segsum_partials(X, ids, n_out, n_quarters, window=128):
    ne, D = X.shape
    DQ = D // n_quarters
    rpt = ((n_out + 15) // 16 + 7) // 8 * 8
    n_out = rpt * 16
    Z = jnp.zeros((n_out, D), jnp.float32)

    @functools.partial(
        pl.kernel,
        out_type=jax.ShapeDtypeStruct((2, n_out, D), jnp.float32),
        mesh=_sc_mesh(),
        scratch_types=[pltpu.VMEM_SHARED((n_out, DQ), jnp.float32)],
        compiler_params=pltpu.CompilerParams(use_tc_tiling_on_sc=False),
    )
    def k(x_hbm, i_hbm, z_hbm, o_hbm, acc_sh):
        c = lax.axis_index("c")
        s = lax.axis_index("s")
        row0 = s * rpt
        for q in range(n_quarters):
            pltpu.sync_copy(z_hbm.at[pl.ds(row0, rpt), pl.ds(0, DQ)],
                            acc_sh.at[pl.ds(row0, rpt), :])
            plsc.subcore_barrier()

            def body(i_vmem, x_vmem):
                pltpu.sync_copy(x_vmem, acc_sh.at[i_vmem], add=True)

            pltpu.emit_pipeline(
                body,
                grid=(ne // window,),
                in_specs=[
                    pl.BlockSpec((window,), lambda i: (i,)),
                    pl.BlockSpec((window, DQ), lambda i, q=q: (i, q)),
                ],
                core_axis_name=("c", "s"),
                dimension_semantics=(pltpu.PARALLEL,),
            )(i_hbm, x_hbm)
            plsc.subcore_barrier()
            pltpu.sync_copy(acc_sh.at[pl.ds(row0, rpt), :],
                            o_hbm.at[c, pl.ds(row0, rpt), pl.ds(q * DQ, DQ)])
            plsc.subcore_barrier()

    return k(X, ids, Z)


def _merge_body(p0_ref, p1_ref, out_ref):
    out_ref[...] = p0_ref[...] + p1_ref[...]


def _merge_add(p0, p1):
    n, d = p0.shape
    return pl.pallas_call(
        _merge_body,
        grid=(pl.cdiv(n, N_NODE_BLK),),
        in_specs=[
            pl.BlockSpec((N_NODE_BLK, d), lambda i: (i, 0)),
            pl.BlockSpec((N_NODE_BLK, d), lambda i: (i, 0)),
        ],
        out_specs=pl.BlockSpec((N_NODE_BLK, d), lambda i: (i, 0)),
        out_shape=jax.ShapeDtypeStruct((n, d), jnp.float32),
    )(p0, p1)


def _node_proj_body(v_ref, w_ref, out_ref):
    out_ref[...] = jnp.dot(v_ref[...], w_ref[...], preferred_element_type=jnp.float32)


def _node_proj(V, W, n_pad):
    n, d = V.shape
    dh = W.shape[1]
    return pl.pallas_call(
        _node_proj_body,
        grid=(pl.cdiv(n_pad, N_NODE_BLK),),
        in_specs=[
            pl.BlockSpec((N_NODE_BLK, d), lambda i: (i, 0)),
            pl.BlockSpec((d, dh), lambda i: (0, 0)),
        ],
        out_specs=pl.BlockSpec((N_NODE_BLK, dh), lambda i: (i, 0)),
        out_shape=jax.ShapeDtypeStruct((n_pad, dh), jnp.float32),
    )(V, W)


def _h0_body(s_ref, e_ref, w_ref, h0_ref, h1_ref):
    h0 = s_ref[...] + jnp.dot(e_ref[...], w_ref[...], preferred_element_type=jnp.float32)
    h0_ref[...] = h0
    h1_ref[...] = jnp.maximum(h0, 0.0)


def _edge_init(S, E, W_ie):
    ne, dh = S.shape
    de = E.shape[1]
    return pl.pallas_call(
        _h0_body,
        grid=(ne // N_EDGE_BLK,),
        in_specs=[
            pl.BlockSpec((N_EDGE_BLK, dh), lambda i: (i, 0)),
            pl.BlockSpec((N_EDGE_BLK, de), lambda i: (i, 0)),
            pl.BlockSpec((de, dh), lambda i: (0, 0)),
        ],
        out_specs=[
            pl.BlockSpec((N_EDGE_BLK, dh), lambda i: (i, 0)),
            pl.BlockSpec((N_EDGE_BLK, dh), lambda i: (i, 0)),
        ],
        out_shape=[
            jax.ShapeDtypeStruct((ne, dh), jnp.float32),
            jax.ShapeDtypeStruct((ne, dh), jnp.float32),
        ],
    )(S, E, W_ie)


def _combine_body(h0_ref, mg_ref, hg_ref, w_ref, out_ref):
    m = mg_ref[...] - hg_ref[...]
    out_ref[...] = jnp.maximum(
        h0_ref[...] + jnp.dot(m, w_ref[...], preferred_element_type=jnp.float32), 0.0
    )


def _edge_combine(H0, Mg, Hg, W_h):
    ne, dh = H0.shape
    return pl.pallas_call(
        _combine_body,
        grid=(ne // N_EDGE_BLK,),
        in_specs=[
            pl.BlockSpec((N_EDGE_BLK, dh), lambda i: (i, 0)),
            pl.BlockSpec((N_EDGE_BLK, dh), lambda i: (i, 0)),
            pl.BlockSpec((N_EDGE_BLK, dh), lambda i: (i, 0)),
            pl.BlockSpec((dh, dh), lambda i: (0, 0)),
        ],
        out_specs=pl.BlockSpec((N_EDGE_BLK, dh), lambda i: (i, 0)),
        out_shape=jax.ShapeDtypeStruct((ne, dh), jnp.float32),
    )(H0, Mg, Hg, W_h)


def _hv_body(v_ref, mv0_ref, mv1_ref, wo1_ref, wo2_ref, bo_ref, wg_ref, bg_ref, out_ref):
    hv = jnp.dot(v_ref[...], wo1_ref[...], preferred_element_type=jnp.float32)
    hv += jnp.dot(mv0_ref[...] + mv1_ref[...], wo2_ref[...], preferred_element_type=jnp.float32)
    hv = jnp.maximum(hv + bo_ref[...], 0.0)
    s = jnp.sum(hv * wg_ref[...], axis=1, keepdims=True) + bg_ref[...]
    out_ref[...] = jax.nn.sigmoid(s) * hv


def _node_out(V, Mv0, Mv1, W_o1, W_o2, b_o, wg_row, b_g):
    n, dv = V.shape
    dh = Mv0.shape[1]
    return pl.pallas_call(
        _hv_body,
        grid=(n // N_NODE_BLK,),
        in_specs=[
            pl.BlockSpec((N_NODE_BLK, dv), lambda i: (i, 0)),
            pl.BlockSpec((N_NODE_BLK, dh), lambda i: (i, 0)),
            pl.BlockSpec((N_NODE_BLK, dh), lambda i: (i, 0)),
            pl.BlockSpec((dv, dh), lambda i: (0, 0)),
            pl.BlockSpec((dh, dh), lambda i: (0, 0)),
            pl.BlockSpec((dh,), lambda i: (0,)),
            pl.BlockSpec((1, dh), lambda i: (0, 0)),
            pl.BlockSpec((1, 1), lambda i: (0, 0)),
        ],
        out_specs=pl.BlockSpec((N_NODE_BLK, dh), lambda i: (i, 0)),
        out_shape=jax.ShapeDtypeStruct((n, dh), jnp.float32),
    )(V, Mv0, Mv1, W_o1, W_o2, b_o, wg_row, b_g)


def _mlp_body(hg0_ref, hg1_ref, vd_ref, lng_ref, lnb_ref, w1_ref, b1_ref, w2_ref, b2_ref, out_ref):
    x = jnp.concatenate([hg0_ref[...] + hg1_ref[...], vd_ref[...]], axis=1)
    mu = jnp.mean(x, axis=-1, keepdims=True)
    var = jnp.mean((x - mu) ** 2, axis=-1, keepdims=True)
    x = (x - mu) * lax.rsqrt(var + 1e-5) * lng_ref[...] + lnb_ref[...]
    x1 = jnp.maximum(jnp.dot(x, w1_ref[...], preferred_element_type=jnp.float32) + b1_ref[...], 0.0)
    out_ref[...] = jnp.dot(x1, w2_ref[...], preferred_element_type=jnp.float32) + b2_ref[...]


def _fusion_mlp(hg0, hg1, V_d, ln_g, ln_b, W1, b1, W2, b2):
    n = V_d.shape[0]
    dh = hg0.shape[1]
    dlm = V_d.shape[1]
    d = dh + dlm
    d_hid = W1.shape[1]
    blk = 200
    W2p = jnp.zeros((d_hid, 128), W2.dtype).at[:, :1].set(W2)
    b2p = jnp.zeros((128,), b2.dtype).at[:1].set(b2)
    out = pl.pallas_call(
        _mlp_body,
        grid=(n // blk,),
        in_specs=[
            pl.BlockSpec((blk, dh), lambda i: (i, 0)),
            pl.BlockSpec((blk, dh), lambda i: (i, 0)),
            pl.BlockSpec((blk, dlm), lambda i: (i, 0)),
            pl.BlockSpec((d,), lambda i: (0,)),
            pl.BlockSpec((d,), lambda i: (0,)),
            pl.BlockSpec((d, d_hid), lambda i: (0, 0)),
            pl.BlockSpec((d_hid,), lambda i: (0,)),
            pl.BlockSpec((d_hid, 128), lambda i: (0, 0)),
            pl.BlockSpec((128,), lambda i: (0,)),
        ],
        out_specs=pl.BlockSpec((blk, 128), lambda i: (i, 0)),
        out_shape=jax.ShapeDtypeStruct((n, 128), jnp.float32),
    )(hg0, hg1, V_d, ln_g, ln_b, W1, b1, W2p, b2p)
    return out[:, :1]


def kernel(V, E, edge_index, rev_edge_index, batch, V_d, W_i, W_h, W_o, b_o,
           w_g, b_g, ln_g, ln_b, W1, b1, W2, b2):
    n_nodes, d_v = V.shape
    src = edge_index[0]
    dst = edge_index[1]
    W_iv = W_i[:d_v]
    W_ie = W_i[d_v:]
    W_o1 = W_o[:d_v]
    W_o2 = W_o[d_v:]

    P = _node_proj(V, W_iv, n_nodes)
    S = _sc_gather_rows(P, src)
    H0, H = _edge_init(S, E, W_ie)

    for _ in range(1, DEPTH):
        Mp = _sc_segsum_partials(H, dst, n_nodes, 4)
        M_node = _merge_add(Mp[0], Mp[1])
        Mg, Hg = _sc_gather_rows2(M_node, src, H, rev_edge_index)
        H = _edge_combine(H0, Mg, Hg, W_h)

    Mvp = _sc_segsum_partials(H, dst, n_nodes, 4)
    weighted = _node_out(V, Mvp[0], Mvp[1], W_o1, W_o2, b_o,
                         w_g.reshape(1, -1), b_g.reshape(1, 1))
    Pp = _sc_segsum_partials(weighted, batch, V_d.shape[0], 1, window=80)
    return _fusion_mlp(Pp[0], Pp[1], V_d, ln_g, ln_b, W1, b1, W2, b2)


# speedup vs baseline: 12.5688x; 1.0400x over previous
import functools

import jax
import jax.numpy as jnp
from jax import lax
from jax.experimental import pallas as pl
from jax.experimental.pallas import tpu as pltpu
from jax.experimental.pallas import tpu_sc as plsc

DEPTH = 3
N_EDGE_BLK = 4000
N_NODE_BLK = 2000


def _sc_mesh():
    return plsc.VectorSubcoreMesh(core_axis_name="c", subcore_axis_name="s")


def _sc_gather_rows(table, idx, window=128):
    B = idx.shape[0]
    D = table.shape[1]
    idx2 = idx.reshape(1, B)

    @functools.partial(
        pl.kernel,
        out_type=jax.ShapeDtypeStruct((B, D), table.dtype),
        mesh=_sc_mesh(),
    )
    def k(table_hbm, i_hbm, o_hbm):
        def body(i_vmem, o_vmem):
            pltpu.sync_copy(table_hbm.at[i_vmem.at[0]], o_vmem)

        pltpu.emit_pipeline(
            body,
            grid=(B // window,),
            in_specs=[pl.BlockSpec((1, window), lambda i: (0, i))],
            out_specs=[pl.BlockSpec((window, D), lambda i: (i, 0))],
            core_axis_name=("c", "s"),
            dimension_semantics=(pltpu.PARALLEL,),
        )(i_hbm, o_hbm)

    return k(table, idx2)


def _sc_gather_rows2(table_a, idx_a, table_b, idx_b, window=128):
    B = idx_a.shape[0]
    D = table_a.shape[1]
    ia2 = idx_a.reshape(1, B)
    ib2 = idx_b.reshape(1, B)

    @functools.partial(
        pl.kernel,
        out_type=(
            jax.ShapeDtypeStruct((B, D), table_a.dtype),
            jax.ShapeDtypeStruct((B, D), table_b.dtype),
        ),
        mesh=_sc_mesh(),
        scratch_types=[pltpu.SemaphoreType.DMA, pltpu.SemaphoreType.DMA],
    )
    def k(ta_hbm, ia_hbm, tb_hbm, ib_hbm, oa_hbm, ob_hbm, sem_a, sem_b):
        def body(ia_vmem, ib_vmem, oa_vmem, ob_vmem):
            ca = pltpu.async_copy(ta_hbm.at[ia_vmem.at[0]], oa_vmem, sem_a)
            cb = pltpu.async_copy(tb_hbm.at[ib_vmem.at[0]], ob_vmem, sem_b)
            ca.wait()
            cb.wait()

        pltpu.emit_pipeline(
            body,
            grid=(B // window,),
            in_specs=[
                pl.BlockSpec((1, window), lambda i: (0, i)),
                pl.BlockSpec((1, window), lambda i: (0, i)),
            ],
            out_specs=[
                pl.BlockSpec((window, D), lambda i: (i, 0)),
                pl.BlockSpec((window, D), lambda i: (i, 0)),
            ],
            core_axis_name=("c", "s"),
            dimension_semantics=(pltpu.PARALLEL,),
        )(ia_hbm, ib_hbm, oa_hbm, ob_hbm)

    return k(table_a, ia2, table_b, ib2)


def _sc_segsum_partials(X, ids, n_out, n_quarters, window=128):
    ne, D = X.shape
    DQ = D // n_quarters
    rpt = ((n_out + 15) // 16 + 7) // 8 * 8
    n_out = rpt * 16
    Z = jnp.zeros((n_out, D), jnp.float32)

    @functools.partial(
        pl.kernel,
        out_type=jax.ShapeDtypeStruct((2, n_out, D), jnp.float32),
        mesh=_sc_mesh(),
        scratch_types=[pltpu.VMEM_SHARED((n_out, DQ), jnp.float32)],
        compiler_params=pltpu.CompilerParams(use_tc_tiling_on_sc=False),
    )
    def k(x_hbm, i_hbm, z_hbm, o_hbm, acc_sh):
        c = lax.axis_index("c")
        s = lax.axis_index("s")
        row0 = s * rpt
        for q in range(n_quarters):
            pltpu.sync_copy(z_hbm.at[pl.ds(row0, rpt), pl.ds(0, DQ)],
                            acc_sh.at[pl.ds(row0, rpt), :])
            plsc.subcore_barrier()

            def body(i_vmem, x_vmem):
                pltpu.sync_copy(x_vmem, acc_sh.at[i_vmem], add=True)

            pltpu.emit_pipeline(
                body,
                grid=(ne // window,),
                in_specs=[
                    pl.BlockSpec((window,), lambda i: (i,)),
                    pl.BlockSpec((window, DQ), lambda i, q=q: (i, q)),
                ],
                core_axis_name=("c", "s"),
                dimension_semantics=(pltpu.PARALLEL,),
            )(i_hbm, x_hbm)
            plsc.subcore_barrier()
            pltpu.sync_copy(acc_sh.at[pl.ds(row0, rpt), :],
                            o_hbm.at[c, pl.ds(row0, rpt), pl.ds(q * DQ, DQ)])
            plsc.subcore_barrier()

    return k(X, ids, Z)


def _merge_body(p0_ref, p1_ref, out_ref):
    out_ref[...] = p0_ref[...] + p1_ref[...]


def _merge_add(p0, p1):
    n, d = p0.shape
    return pl.pallas_call(
        _merge_body,
        grid=(pl.cdiv(n, N_NODE_BLK),),
        in_specs=[
            pl.BlockSpec((N_NODE_BLK, d), lambda i: (i, 0)),
            pl.BlockSpec((N_NODE_BLK, d), lambda i: (i, 0)),
        ],
        out_specs=pl.BlockSpec((N_NODE_BLK, d), lambda i: (i, 0)),
        out_shape=jax.ShapeDtypeStruct((n, d), jnp.float32),
    )(p0, p1)


def _node_proj_body(v_ref, w_ref, out_ref):
    out_ref[...] = jnp.dot(v_ref[...], w_ref[...], preferred_element_type=jnp.float32)


def _node_proj(V, W, n_pad):
    n, d = V.shape
    dh = W.shape[1]
    return pl.pallas_call(
        _node_proj_body,
        grid=(pl.cdiv(n_pad, N_NODE_BLK),),
        in_specs=[
            pl.BlockSpec((N_NODE_BLK, d), lambda i: (i, 0)),
            pl.BlockSpec((d, dh), lambda i: (0, 0)),
        ],
        out_specs=pl.BlockSpec((N_NODE_BLK, dh), lambda i: (i, 0)),
        out_shape=jax.ShapeDtypeStruct((n_pad, dh), jnp.float32),
    )(V, W)


def _h0_body(s_ref, e_ref, w_ref, h0_ref, h1_ref):
    h0 = s_ref[...] + jnp.dot(e_ref[...], w_ref[...], preferred_element_type=jnp.float32)
    h0_ref[...] = h0
    h1_ref[...] = jnp.maximum(h0, 0.0)


def _edge_init(S, E, W_ie):
    ne, dh = S.shape
    de = E.shape[1]
    return pl.pallas_call(
        _h0_body,
        grid=(ne // N_EDGE_BLK,),
        in_specs=[
            pl.BlockSpec((N_EDGE_BLK, dh), lambda i: (i, 0)),
            pl.BlockSpec((N_EDGE_BLK, de), lambda i: (i, 0)),
            pl.BlockSpec((de, dh), lambda i: (0, 0)),
        ],
        out_specs=[
            pl.BlockSpec((N_EDGE_BLK, dh), lambda i: (i, 0)),
            pl.BlockSpec((N_EDGE_BLK, dh), lambda i: (i, 0)),
        ],
        out_shape=[
            jax.ShapeDtypeStruct((ne, dh), jnp.float32),
            jax.ShapeDtypeStruct((ne, dh), jnp.float32),
        ],
    )(S, E, W_ie)


def _combine_body(h0_ref, mg_ref, hg_ref, w_ref, out_ref):
    m = mg_ref[...] - hg_ref[...]
    out_ref[...] = jnp.maximum(
        h0_ref[...] + jnp.dot(m, w_ref[...], preferred_element_type=jnp.float32), 0.0
    )


def _edge_combine(H0, Mg, Hg, W_h):
    ne, dh = H0.shape
    return pl.pallas_call(
        _combine_body,
        grid=(ne // N_EDGE_BLK,),
        in_specs=[
            pl.BlockSpec((N_EDGE_BLK, dh), lambda i: (i, 0)),
            pl.BlockSpec((N_EDGE_BLK, dh), lambda i: (i, 0)),
            pl.BlockSpec((N_EDGE_BLK, dh), lambda i: (i, 0)),
            pl.BlockSpec((dh, dh), lambda i: (0, 0)),
        ],
        out_specs=pl.BlockSpec((N_EDGE_BLK, dh), lambda i: (i, 0)),
        out_shape=jax.ShapeDtypeStruct((ne, dh), jnp.float32),
    )(H0, Mg, Hg, W_h)


def _hv_body(v_ref, mv0_ref, mv1_ref, wo1_ref, wo2_ref, bo_ref, wg_ref, bg_ref, out_ref):
    hv = jnp.dot(v_ref[...], wo1_ref[...], preferred_element_type=jnp.float32)
    hv += jnp.dot(mv0_ref[...] + mv1_ref[...], wo2_ref[...], preferred_element_type=jnp.float32)
    hv = jnp.maximum(hv + bo_ref[...], 0.0)
    s = jnp.sum(hv * wg_ref[...], axis=1, keepdims=True) + bg_ref[...]
    out_ref[...] = jax.nn.sigmoid(s) * hv


def _node_out(V, Mv0, Mv1, W_o1, W_o2, b_o, wg_row, b_g):
    n, dv = V.shape
    dh = Mv0.shape[1]
    return pl.pallas_call(
        _hv_body,
        grid=(n // N_NODE_BLK,),
        in_specs=[
            pl.BlockSpec((N_NODE_BLK, dv), lambda i: (i, 0)),
            pl.BlockSpec((N_NODE_BLK, dh), lambda i: (i, 0)),
            pl.BlockSpec((N_NODE_BLK, dh), lambda i: (i, 0)),
            pl.BlockSpec((dv, dh), lambda i: (0, 0)),
            pl.BlockSpec((dh, dh), lambda i: (0, 0)),
            pl.BlockSpec((dh,), lambda i: (0,)),
            pl.BlockSpec((1, dh), lambda i: (0, 0)),
            pl.BlockSpec((1, 1), lambda i: (0, 0)),
        ],
        out_specs=pl.BlockSpec((N_NODE_BLK, dh), lambda i: (i, 0)),
        out_shape=jax.ShapeDtypeStruct((n, dh), jnp.float32),
    )(V, Mv0, Mv1, W_o1, W_o2, b_o, wg_row, b_g)


def _mlp_body(hg0_ref, hg1_ref, vd_ref, lng_ref, lnb_ref, w1_ref, b1_ref, w2_ref, b2_ref, out_ref):
    x = jnp.concatenate([hg0_ref[...] + hg1_ref[...], vd_ref[...]], axis=1)
    mu = jnp.mean(x, axis=-1, keepdims=True)
    var = jnp.mean((x - mu) ** 2, axis=-1, keepdims=True)
    x = (x - mu) * lax.rsqrt(var + 1e-5) * lng_ref[...] + lnb_ref[...]
    x1 = jnp.maximum(jnp.dot(x, w1_ref[...], preferred_element_type=jnp.float32) + b1_ref[...], 0.0)
    out_ref[...] = jnp.dot(x1, w2_ref[...], preferred_element_type=jnp.float32) + b2_ref[...]


def _fusion_mlp(hg0, hg1, V_d, ln_g, ln_b, W1, b1, W2, b2):
    n = V_d.shape[0]
    dh = hg0.shape[1]
    dlm = V_d.shape[1]
    d = dh + dlm
    d_hid = W1.shape[1]
    blk = 200
    W2p = jnp.zeros((d_hid, 128), W2.dtype).at[:, :1].set(W2)
    b2p = jnp.zeros((128,), b2.dtype).at[:1].set(b2)
    out = pl.pallas_call(
        _mlp_body,
        grid=(n // blk,),
        in_specs=[
            pl.BlockSpec((blk, dh), lambda i: (i, 0)),
            pl.BlockSpec((blk, dh), lambda i: (i, 0)),
            pl.BlockSpec((blk, dlm), lambda i: (i, 0)),
            pl.BlockSpec((d,), lambda i: (0,)),
            pl.BlockSpec((d,), lambda i: (0,)),
            pl.BlockSpec((d, d_hid), lambda i: (0, 0)),
            pl.BlockSpec((d_hid,), lambda i: (0,)),
            pl.BlockSpec((d_hid, 128), lambda i: (0, 0)),
            pl.BlockSpec((128,), lambda i: (0,)),
        ],
        out_specs=pl.BlockSpec((blk, 128), lambda i: (i, 0)),
        out_shape=jax.ShapeDtypeStruct((n, 128), jnp.float32),
    )(hg0, hg1, V_d, ln_g, ln_b, W1, b1, W2p, b2p)
    return out[:, :1]


def kernel(V, E, edge_index, rev_edge_index, batch, V_d, W_i, W_h, W_o, b_o,
           w_g, b_g, ln_g, ln_b, W1, b1, W2, b2):
    n_nodes, d_v = V.shape
    src = edge_index[0]
    dst = edge_index[1]
    W_iv = W_i[:d_v]
    W_ie = W_i[d_v:]
    W_o1 = W_o[:d_v]
    W_o2 = W_o[d_v:]

    P = _node_proj(V, W_iv, n_nodes)
    S = _sc_gather_rows(P, src)
    H0, H = _edge_init(S, E, W_ie)

    for _ in range(1, DEPTH):
        Mp = _sc_segsum_partials(H, dst, n_nodes, 4)
        M_node = _merge_add(Mp[0], Mp[1])
        Mg, Hg = _sc_gather_rows2(M_node, src, H, rev_edge_index)
        H = _edge_combine(H0, Mg, Hg, W_h)

    Mvp = _sc_segsum_partials(H, dst, n_nodes, 4)
    weighted = _node_out(V, Mvp[0], Mvp[1], W_o1, W_o2, b_o,
                         w_g.reshape(1, -1), b_g.reshape(1, 1))
    Pp = _sc_segsum_partials(weighted, batch, V_d.shape[0], 1, window=80)
    return _fusion_mlp(Pp[0], Pp[1], V_d, ln_g, ln_b, W1, b1, W2, b2)
